# Initial kernel scaffold; baseline (speedup 1.0000x reference)
import jax
import jax.numpy as jnp
from jax.experimental import pallas as pl


def kernel(x, edge_index, W1_0, b1_0, W2_0, b2_0, Ws1_0, Ws2_0, W1_1, b1_1, W2_1, b2_1, Ws1_1, Ws2_1):
    raise NotImplementedError("write your pallas kernel here")


# speedup vs baseline: 1.9802x; 1.9802x over previous
import functools

import jax
import jax.numpy as jnp
from jax import lax
from jax.experimental import pallas as pl
from jax.experimental.pallas import tpu as pltpu
from jax.experimental.pallas import tpu_sc as plsc

N = 10000
E = 320000
R = 3
D = 128
A = 16

NC = 2
NS = 16
NW = NC * NS

CH = 128
NCHUNK = 80
EPT = CH * NCHUNK
E_PAD = EPT * NW

N_ACC = 10240
ZROWS = N_ACC // NS
OROWS = N_ACC // NS


def _sc_segsum(h, srcq, dstq, zeros):
  mesh = plsc.VectorSubcoreMesh(core_axis_name="c", subcore_axis_name="s")

  @functools.partial(
      pl.kernel,
      mesh=mesh,
      out_type=jax.ShapeDtypeStruct((R, NC, N_ACC, D), jnp.float32),
      scratch_types=[
          pltpu.VMEM_SHARED((N_ACC, D), jnp.float32),
          pltpu.VMEM((NCHUNK, CH), jnp.int32),
          pltpu.VMEM((NCHUNK, CH), jnp.int32),
          pltpu.VMEM((CH, D), jnp.float32),
          pltpu.SemaphoreType.DMA,
      ],
  )
  def k(h_hbm, src_hbm, dst_hbm, z_hbm, out_hbm, acc, srcv, dstv, rows, sem):
    cid = lax.axis_index("c")
    sid = lax.axis_index("s")
    wid = sid * NC + cid

    for r in range(R):
      pltpu.sync_copy(z_hbm, acc.at[pl.ds(sid * ZROWS, ZROWS)])
      pltpu.sync_copy(src_hbm.at[r, wid], srcv)
      pltpu.sync_copy(dst_hbm.at[r, wid], dstv)
      plsc.subcore_barrier()

      def chunk(j, _):
        pltpu.async_copy(h_hbm.at[srcv.at[j]], rows, sem).wait()
        pltpu.sync_copy(rows, acc.at[dstv.at[j]], add=True)
        return _

      lax.fori_loop(0, NCHUNK, chunk, None)
      plsc.subcore_barrier()
      pltpu.sync_copy(
          acc.at[pl.ds(sid * OROWS, OROWS)],
          out_hbm.at[r, cid, pl.ds(sid * OROWS, OROWS)],
      )
      if r + 1 < R:
        plsc.subcore_barrier()

  return k(h, srcq, dstq, zeros)


def _tc_layer_body(h_ref, agg_ref, w1_ref, b1_ref, w2_ref, b2_ref,
                   ws1_ref, ws2_ref, out_ref):
  hb = h_ref[...]
  w1 = w1_ref[...]
  b1 = b1_ref[...]
  w2 = w2_ref[...]
  b2 = b2_ref[...]
  zs = []
  ls = []
  for r in range(R):
    z = hb + agg_ref[r, 0] + agg_ref[r, 1]
    z = jnp.maximum(jnp.dot(z, w1, preferred_element_type=jnp.float32) + b1, 0.0)
    z = jnp.maximum(jnp.dot(z, w2, preferred_element_type=jnp.float32) + b2, 0.0)
    t = jnp.tanh(jnp.dot(z, ws1_ref[r], preferred_element_type=jnp.float32))
    l = jnp.sum(t * ws2_ref[r], axis=1, keepdims=True)
    zs.append(z)
    ls.append(l)
  m = jnp.maximum(jnp.maximum(ls[0], ls[1]), ls[2])
  es = [jnp.exp(l - m) for l in ls]
  denom = es[0] + es[1] + es[2]
  out_ref[...] = (es[0] * zs[0] + es[1] * zs[1] + es[2] * zs[2]) / denom


def _tc_layer(h, agg, w1, b1, w2, b2, ws1p, ws2p, blk, grid):
  return pl.pallas_call(
      _tc_layer_body,
      grid=(grid,),
      in_specs=[
          pl.BlockSpec((blk, D), lambda i: (i, 0)),
          pl.BlockSpec((R, NC, blk, D), lambda i: (0, 0, i, 0)),
          pl.BlockSpec((D, D), lambda i: (0, 0)),
          pl.BlockSpec((1, D), lambda i: (0, 0)),
          pl.BlockSpec((D, D), lambda i: (0, 0)),
          pl.BlockSpec((1, D), lambda i: (0, 0)),
          pl.BlockSpec((R, D, D), lambda i: (0, 0, 0)),
          pl.BlockSpec((R, 1, D), lambda i: (0, 0, 0)),
      ],
      out_specs=pl.BlockSpec((blk, D), lambda i: (i, 0)),
      out_shape=jax.ShapeDtypeStruct((grid * blk, D), jnp.float32),
  )(h, agg, w1, b1, w2, b2, ws1p, ws2p)


def kernel(x, edge_index, W1_0, b1_0, W2_0, b2_0, Ws1_0, Ws2_0,
           W1_1, b1_1, W2_1, b2_1, Ws1_1, Ws2_1):
  src = edge_index[:, 0, :]
  dst = edge_index[:, 1, :]
  pad = E_PAD - E
  src = jnp.concatenate([src, jnp.zeros((R, pad), jnp.int32)], axis=1)
  dst = jnp.concatenate([dst, jnp.full((R, pad), N, jnp.int32)], axis=1)
  srcq = src.reshape(R, NW, NCHUNK, CH)
  dstq = dst.reshape(R, NW, NCHUNK, CH)
  zeros = jnp.zeros((ZROWS, D), jnp.float32)

  blk, grid = 1000, 10

  h = x
  for (w1, b1, w2, b2, ws1, ws2) in (
      (W1_0, b1_0, W2_0, b2_0, Ws1_0, Ws2_0),
      (W1_1, b1_1, W2_1, b2_1, Ws1_1, Ws2_1),
  ):
    agg = _sc_segsum(h, srcq, dstq, zeros)
    ws1p = jnp.pad(ws1, ((0, 0), (0, 0), (0, D - A)))
    ws2p = jnp.pad(ws2[:, :, 0], ((0, 0), (0, D - A))).reshape(R, 1, D)
    h = _tc_layer(h, agg, w1, b1.reshape(1, D), w2, b2.reshape(1, D),
                  ws1p, ws2p, blk, grid)
  return h


# speedup vs baseline: 2.2134x; 1.1177x over previous
import functools

import jax
import jax.numpy as jnp
from jax import lax
from jax.experimental import pallas as pl
from jax.experimental.pallas import tpu as pltpu
from jax.experimental.pallas import tpu_sc as plsc

N = 10000
E = 320000
R = 3
D = 128
A = 16

NC = 2
NS = 16
NW = NC * NS

CH = 128
NCHUNK = 80
EPT = CH * NCHUNK
E_PAD = EPT * NW

N_ACC = 10240
ZROWS = N_ACC // NS
OROWS = N_ACC // NS


def _sc_segsum(h, srcq, dstq, zeros):
  mesh = plsc.VectorSubcoreMesh(core_axis_name="c", subcore_axis_name="s")

  @functools.partial(
      pl.kernel,
      mesh=mesh,
      out_type=jax.ShapeDtypeStruct((R, NC, N_ACC, D), jnp.float32),
      scratch_types=[
          pltpu.VMEM_SHARED((N_ACC, D), jnp.float32),
          pltpu.VMEM((NCHUNK // 2, CH), jnp.int32),
          pltpu.VMEM((NCHUNK // 2, CH), jnp.int32),
          pltpu.VMEM((CH, D), jnp.float32),
          pltpu.VMEM((CH, D), jnp.float32),
          pltpu.SemaphoreType.DMA,
          pltpu.SemaphoreType.DMA,
      ],
  )
  def k(h_hbm, src_hbm, dst_hbm, z_hbm, out_hbm, acc, srcv, dstv,
        rows0, rows1, s0, s1):
    cid = lax.axis_index("c")
    sid = lax.axis_index("s")
    wid = sid * NC + cid

    for r in range(R):
      pltpu.sync_copy(z_hbm, acc.at[pl.ds(sid * ZROWS, ZROWS)])
      plsc.subcore_barrier()

      half = NCHUNK // 2
      for st in range(2):
        pltpu.sync_copy(src_hbm.at[r, wid, pl.ds(st * half, half)], srcv)
        pltpu.sync_copy(dst_hbm.at[r, wid, pl.ds(st * half, half)], dstv)

        pltpu.async_copy(h_hbm.at[srcv.at[0]], rows0, s0)

        def chunk_pair(i, _):
          pltpu.async_copy(h_hbm.at[srcv.at[2 * i + 1]], rows1, s1)
          pltpu.make_async_copy(h_hbm.at[srcv.at[0]], rows0, s0).wait()
          pltpu.sync_copy(rows0, acc.at[dstv.at[2 * i]], add=True)
          nxt = lax.rem(2 * i + 2, half)
          pltpu.async_copy(h_hbm.at[srcv.at[nxt]], rows0, s0)
          pltpu.make_async_copy(h_hbm.at[srcv.at[0]], rows1, s1).wait()
          pltpu.sync_copy(rows1, acc.at[dstv.at[2 * i + 1]], add=True)
          return _

        lax.fori_loop(0, half // 2, chunk_pair, None)
        pltpu.make_async_copy(h_hbm.at[srcv.at[0]], rows0, s0).wait()
      plsc.subcore_barrier()
      pltpu.sync_copy(
          acc.at[pl.ds(sid * OROWS, OROWS)],
          out_hbm.at[r, cid, pl.ds(sid * OROWS, OROWS)],
      )
      if r + 1 < R:
        plsc.subcore_barrier()

  return k(h, srcq, dstq, zeros)


def _tc_layer_body(h_ref, agg_ref, w1_ref, b1_ref, w2_ref, b2_ref,
                   ws1_ref, ws2_ref, out_ref):
  hb = h_ref[...]
  w1 = w1_ref[...]
  b1 = b1_ref[...]
  w2 = w2_ref[...]
  b2 = b2_ref[...]
  zs = []
  ls = []
  for r in range(R):
    z = hb + agg_ref[r, 0] + agg_ref[r, 1]
    z = jnp.maximum(jnp.dot(z, w1, preferred_element_type=jnp.float32) + b1, 0.0)
    z = jnp.maximum(jnp.dot(z, w2, preferred_element_type=jnp.float32) + b2, 0.0)
    t = jnp.tanh(jnp.dot(z, ws1_ref[r], preferred_element_type=jnp.float32))
    l = jnp.sum(t * ws2_ref[r], axis=1, keepdims=True)
    zs.append(z)
    ls.append(l)
  m = jnp.maximum(jnp.maximum(ls[0], ls[1]), ls[2])
  es = [jnp.exp(l - m) for l in ls]
  denom = es[0] + es[1] + es[2]
  out_ref[...] = (es[0] * zs[0] + es[1] * zs[1] + es[2] * zs[2]) / denom


def _tc_layer(h, agg, w1, b1, w2, b2, ws1p, ws2p, blk, grid):
  return pl.pallas_call(
      _tc_layer_body,
      grid=(grid,),
      in_specs=[
          pl.BlockSpec((blk, D), lambda i: (i, 0)),
          pl.BlockSpec((R, NC, blk, D), lambda i: (0, 0, i, 0)),
          pl.BlockSpec((D, D), lambda i: (0, 0)),
          pl.BlockSpec((1, D), lambda i: (0, 0)),
          pl.BlockSpec((D, D), lambda i: (0, 0)),
          pl.BlockSpec((1, D), lambda i: (0, 0)),
          pl.BlockSpec((R, D, D), lambda i: (0, 0, 0)),
          pl.BlockSpec((R, 1, D), lambda i: (0, 0, 0)),
      ],
      out_specs=pl.BlockSpec((blk, D), lambda i: (i, 0)),
      out_shape=jax.ShapeDtypeStruct((grid * blk, D), jnp.float32),
  )(h, agg, w1, b1, w2, b2, ws1p, ws2p)


def kernel(x, edge_index, W1_0, b1_0, W2_0, b2_0, Ws1_0, Ws2_0,
           W1_1, b1_1, W2_1, b2_1, Ws1_1, Ws2_1):
  src = edge_index[:, 0, :]
  dst = edge_index[:, 1, :]
  pad = E_PAD - E
  src = jnp.concatenate([src, jnp.zeros((R, pad), jnp.int32)], axis=1)
  dst = jnp.concatenate([dst, jnp.full((R, pad), N, jnp.int32)], axis=1)
  srcq = src.reshape(R, NW, NCHUNK, CH)
  dstq = dst.reshape(R, NW, NCHUNK, CH)
  zeros = jnp.zeros((ZROWS, D), jnp.float32)

  blk, grid = 1000, 10

  h = x
  for (w1, b1, w2, b2, ws1, ws2) in (
      (W1_0, b1_0, W2_0, b2_0, Ws1_0, Ws2_0),
      (W1_1, b1_1, W2_1, b2_1, Ws1_1, Ws2_1),
  ):
    agg = _sc_segsum(h, srcq, dstq, zeros)
    ws1p = jnp.pad(ws1, ((0, 0), (0, 0), (0, D - A)))
    ws2p = jnp.pad(ws2[:, :, 0], ((0, 0), (0, D - A))).reshape(R, 1, D)
    h = _tc_layer(h, agg, w1, b1.reshape(1, D), w2, b2.reshape(1, D),
                  ws1p, ws2p, blk, grid)
  return h


# speedup vs baseline: 5.7345x; 2.5908x over previous
import functools

import jax
import jax.numpy as jnp
from jax import lax
from jax.experimental import pallas as pl
from jax.experimental.pallas import tpu as pltpu
from jax.experimental.pallas import tpu_sc as plsc

N = 10000
E = 320000
R = 3
D = 128
A = 16

NC = 2
NS = 16
NW = NC * NS

CH = 128
NCHUNK = 80
STCH = NCHUNK // 2
EPT = CH * NCHUNK
E_PAD = EPT * NW

DH = D // 2
N_PAD = 10240
ZROWS = N_PAD // NS
NBUF = 2


def _sc_segsum(hhalves, srcq, dstq, zeros):
  mesh = plsc.VectorSubcoreMesh(core_axis_name="c", subcore_axis_name="s")

  @functools.partial(
      pl.kernel,
      mesh=mesh,
      out_type=jax.ShapeDtypeStruct((2, R, NC, N_PAD, DH), jnp.float32),
      scratch_types=[
          pltpu.VMEM_SHARED((N_PAD, DH), jnp.float32),
          pltpu.VMEM_SHARED((N_PAD, DH), jnp.float32),
          pltpu.VMEM((STCH, CH), jnp.int32),
          pltpu.VMEM((STCH, CH), jnp.int32),
          pltpu.VMEM((NBUF, CH, DH), jnp.float32),
          pltpu.SemaphoreType.DMA,
          pltpu.SemaphoreType.DMA,
      ],
  )
  def k(h_hbm, src_hbm, dst_hbm, z_hbm, out_hbm, h_sp, acc, srcv, dstv,
        rows, s0, s1):
    cid = lax.axis_index("c")
    sid = lax.axis_index("s")
    wid = sid * NC + cid
    sems = (s0, s1)

    def gather(j, b):
      pltpu.async_copy(h_sp.at[srcv.at[j]], rows.at[b], sems[b])

    def wait(b):
      pltpu.make_async_copy(h_sp.at[srcv.at[0]], rows.at[b], sems[b]).wait()

    for c in range(2):
      pltpu.sync_copy(h_hbm.at[c, pl.ds(sid * ZROWS, ZROWS)],
                      h_sp.at[pl.ds(sid * ZROWS, ZROWS)])
      for r in range(R):
        pltpu.sync_copy(z_hbm, acc.at[pl.ds(sid * ZROWS, ZROWS)])
        plsc.subcore_barrier()

        for st in range(2):
          pltpu.sync_copy(src_hbm.at[r, wid, pl.ds(st * STCH, STCH)], srcv)
          pltpu.sync_copy(dst_hbm.at[r, wid, pl.ds(st * STCH, STCH)], dstv)

          for b in range(NBUF - 1):
            gather(b, b)

          def step(i, _):
            for b in range(NBUF):
              j = NBUF * i + b
              gather(lax.rem(j + NBUF - 1, STCH), (b + NBUF - 1) % NBUF)
              wait(b)
              pltpu.sync_copy(rows.at[b], acc.at[dstv.at[j]], add=True)
            return _

          lax.fori_loop(0, STCH // NBUF, step, None)
          for b in range(NBUF - 1):
            wait(b)
        plsc.subcore_barrier()
        pltpu.sync_copy(
            acc.at[pl.ds(sid * ZROWS, ZROWS)],
            out_hbm.at[c, r, cid, pl.ds(sid * ZROWS, ZROWS)],
        )

  return k(hhalves, srcq, dstq, zeros)


def _tc_layer_body(h_ref, agg_ref, w1_ref, b1_ref, w2_ref, b2_ref,
                   ws1_ref, ws2_ref, out_ref):
  hb = h_ref[...]
  w1 = w1_ref[...]
  b1 = b1_ref[...]
  w2 = w2_ref[...]
  b2 = b2_ref[...]
  zs = []
  ls = []
  for r in range(R):
    agg = jnp.concatenate(
        [agg_ref[0, r, 0] + agg_ref[0, r, 1],
         agg_ref[1, r, 0] + agg_ref[1, r, 1]], axis=1)
    z = hb + agg
    z = jnp.maximum(jnp.dot(z, w1, preferred_element_type=jnp.float32) + b1, 0.0)
    z = jnp.maximum(jnp.dot(z, w2, preferred_element_type=jnp.float32) + b2, 0.0)
    t = jnp.tanh(jnp.dot(z, ws1_ref[r], preferred_element_type=jnp.float32))
    l = jnp.sum(t * ws2_ref[r], axis=1, keepdims=True)
    zs.append(z)
    ls.append(l)
  m = jnp.maximum(jnp.maximum(ls[0], ls[1]), ls[2])
  es = [jnp.exp(l - m) for l in ls]
  denom = es[0] + es[1] + es[2]
  out_ref[...] = (es[0] * zs[0] + es[1] * zs[1] + es[2] * zs[2]) / denom


def _tc_layer(h, agg, w1, b1, w2, b2, ws1p, ws2p, blk, grid):
  return pl.pallas_call(
      _tc_layer_body,
      grid=(grid,),
      in_specs=[
          pl.BlockSpec((blk, D), lambda i: (i, 0)),
          pl.BlockSpec((2, R, NC, blk, DH), lambda i: (0, 0, 0, i, 0)),
          pl.BlockSpec((D, D), lambda i: (0, 0)),
          pl.BlockSpec((1, D), lambda i: (0, 0)),
          pl.BlockSpec((D, D), lambda i: (0, 0)),
          pl.BlockSpec((1, D), lambda i: (0, 0)),
          pl.BlockSpec((R, D, D), lambda i: (0, 0, 0)),
          pl.BlockSpec((R, 1, D), lambda i: (0, 0, 0)),
      ],
      out_specs=pl.BlockSpec((blk, D), lambda i: (i, 0)),
      out_shape=jax.ShapeDtypeStruct((grid * blk, D), jnp.float32),
  )(h, agg, w1, b1, w2, b2, ws1p, ws2p)


def kernel(x, edge_index, W1_0, b1_0, W2_0, b2_0, Ws1_0, Ws2_0,
           W1_1, b1_1, W2_1, b2_1, Ws1_1, Ws2_1):
  src = edge_index[:, 0, :]
  dst = edge_index[:, 1, :]
  pad = E_PAD - E
  src = jnp.concatenate([src, jnp.zeros((R, pad), jnp.int32)], axis=1)
  dst = jnp.concatenate([dst, jnp.full((R, pad), N, jnp.int32)], axis=1)
  srcq = src.reshape(R, NW, NCHUNK, CH)
  dstq = dst.reshape(R, NW, NCHUNK, CH)
  zeros = jnp.zeros((ZROWS, DH), jnp.float32)

  blk, grid = 1000, 10

  h = x
  for (w1, b1, w2, b2, ws1, ws2) in (
      (W1_0, b1_0, W2_0, b2_0, Ws1_0, Ws2_0),
      (W1_1, b1_1, W2_1, b2_1, Ws1_1, Ws2_1),
  ):
    hhalves = jnp.pad(h, ((0, N_PAD - N), (0, 0)))
    hhalves = hhalves.reshape(N_PAD, 2, DH).transpose(1, 0, 2)
    agg = _sc_segsum(hhalves, srcq, dstq, zeros)
    ws1p = jnp.pad(ws1, ((0, 0), (0, 0), (0, D - A)))
    ws2p = jnp.pad(ws2[:, :, 0], ((0, 0), (0, D - A))).reshape(R, 1, D)
    h = _tc_layer(h, agg, w1, b1.reshape(1, D), w2, b2.reshape(1, D),
                  ws1p, ws2p, blk, grid)
  return h
